# Initial kernel scaffold; baseline (speedup 1.0000x reference)
import jax
import jax.numpy as jnp
from jax.experimental import pallas as pl


def kernel(seq, W):
    raise NotImplementedError("write your pallas kernel here")


# speedup vs baseline: 1.5688x; 1.5688x over previous
import functools

import jax
import jax.numpy as jnp
from jax import lax
from jax.experimental import pallas as pl
from jax.experimental.pallas import tpu as pltpu
from jax.experimental.pallas import tpu_sc as plsc

_VOCAB = 26
_DIM = 64
_LANES = 16
_CHUNK = 128


def kernel(seq, W):
    n = seq.shape[0]
    mesh = plsc.VectorSubcoreMesh(core_axis_name="c", subcore_axis_name="s")

    @functools.partial(
        pl.kernel,
        out_type=jax.ShapeDtypeStruct((n, _DIM), W.dtype),
        mesh=mesh,
        compiler_params=pltpu.CompilerParams(use_tc_tiling_on_sc=False),
    )
    def emb(seq_hbm, w_hbm, out_hbm):
        def body(idx_vmem, out_vmem):
            @pl.loop(0, _CHUNK, step=_LANES)
            def _(c):
                sl = pl.ds(c, _LANES)
                v = idx_vmem.at[0][sl]
                idx_vmem.at[0][sl] = lax.rem(v, _VOCAB)

            pltpu.sync_copy(w_hbm.at[idx_vmem.at[0]], out_vmem)

        pltpu.emit_pipeline(
            body,
            grid=(n // _CHUNK,),
            in_specs=[pl.BlockSpec((1, _CHUNK), lambda i: (0, i))],
            out_specs=[pl.BlockSpec((_CHUNK, _DIM), lambda i: (i, 0))],
            core_axis_name=("c", "s"),
            dimension_semantics=(pltpu.PARALLEL,),
        )(seq_hbm, out_hbm)

    return emb(seq.reshape(1, n), W)


# speedup vs baseline: 2.7508x; 1.7535x over previous
import dataclasses
import functools

import jax
import jax.numpy as jnp
from jax.experimental import pallas as pl
from jax.experimental.pallas import tpu as pltpu
from jax.experimental.pallas import tpu_sc as plsc

_VOCAB = 26
_DIM = 64
_LANES = 16
_GATHER = 128
_CHUNK = 256


def _mod26(v):
    s = (v >= 26).astype(jnp.int32)
    s += (v >= 52).astype(jnp.int32)
    s += (v >= 78).astype(jnp.int32)
    s += (v >= 104).astype(jnp.int32)
    return v - 26 * s


def kernel(seq, W):
    n = seq.shape[0]
    npair = n // 2
    seq2 = seq.reshape(npair, 2)
    a = seq2[:, 0].reshape(1, npair)
    b = seq2[:, 1].reshape(1, npair)
    w2 = jnp.concatenate(
        [jnp.repeat(W, _VOCAB, axis=0), jnp.tile(W, (_VOCAB, 1))], axis=1
    )

    mesh = plsc.VectorSubcoreMesh(core_axis_name="c", subcore_axis_name="s")
    cp = pltpu.CompilerParams()
    if "needs_layout_passes" in pltpu.CompilerParams.__dataclass_fields__:
        cp = dataclasses.replace(cp, needs_layout_passes=False)

    @functools.partial(
        pl.kernel,
        out_type=jax.ShapeDtypeStruct((npair, 2 * _DIM), W.dtype),
        mesh=mesh,
        scratch_types=[pltpu.SemaphoreType.DMA],
        compiler_params=cp,
    )
    def emb(a_hbm, b_hbm, w2_hbm, out_hbm, sem):
        def body(a_vmem, b_vmem, out_vmem):
            @pl.loop(0, _CHUNK, step=_LANES)
            def _(c):
                sl = pl.ds(c, _LANES)
                va = a_vmem.at[0][sl]
                vb = b_vmem.at[0][sl]
                a_vmem.at[0][sl] = _mod26(va) * _VOCAB + _mod26(vb)

            copies = []
            for j in range(_CHUNK // _GATHER):
                sl = pl.ds(j * _GATHER, _GATHER)
                copies.append(
                    pltpu.async_copy(
                        w2_hbm.at[a_vmem.at[0, sl]], out_vmem.at[sl], sem
                    )
                )
            for cp in copies:
                cp.wait()

        pltpu.emit_pipeline(
            body,
            grid=(npair // _CHUNK,),
            in_specs=[
                pl.BlockSpec((1, _CHUNK), lambda i: (0, i)),
                pl.BlockSpec((1, _CHUNK), lambda i: (0, i)),
            ],
            out_specs=[pl.BlockSpec((_CHUNK, 2 * _DIM), lambda i: (i, 0))],
            core_axis_name=("c", "s"),
            dimension_semantics=(pltpu.PARALLEL,),
        )(a_hbm, b_hbm, out_hbm)

    out2 = emb(a, b, w2)
    return out2.reshape(n, _DIM)


# speedup vs baseline: 4.1413x; 1.5055x over previous
import dataclasses
import functools

import jax
import jax.numpy as jnp
from jax.experimental import pallas as pl
from jax.experimental.pallas import tpu as pltpu
from jax.experimental.pallas import tpu_sc as plsc

_VOCAB = 26
_DIM = 64
_LANES = 16
_GATHER = 128
_CHUNK = 256


def _mod26(v):
    s = (v >= 26).astype(jnp.int32)
    s += (v >= 52).astype(jnp.int32)
    s += (v >= 78).astype(jnp.int32)
    s += (v >= 104).astype(jnp.int32)
    return v - 26 * s


def kernel(seq, W):
    n = seq.shape[0]
    npair = n // 2
    seq32 = seq
    w2 = jnp.concatenate(
        [jnp.repeat(W, _VOCAB, axis=0), jnp.tile(W, (_VOCAB, 1))], axis=1
    )

    mesh = plsc.VectorSubcoreMesh(core_axis_name="c", subcore_axis_name="s")
    cp = pltpu.CompilerParams()
    if "needs_layout_passes" in pltpu.CompilerParams.__dataclass_fields__:
        cp = dataclasses.replace(cp, needs_layout_passes=False)

    @functools.partial(
        pl.kernel,
        out_type=jax.ShapeDtypeStruct((npair, 2 * _DIM), W.dtype),
        mesh=mesh,
        scratch_types=[
            pltpu.VMEM((_CHUNK,), jnp.int32),
            pltpu.SemaphoreType.DMA,
        ],
        compiler_params=cp,
    )
    def emb(seq_hbm, w2_hbm, out_hbm, pair_ref, sem):
        def body(seq_vmem, out_vmem):
            @pl.loop(0, _CHUNK, step=_LANES)
            def _(c):
                ii = jax.lax.iota(jnp.int32, _LANES) * 2 + 2 * c
                a = plsc.load_gather(seq_vmem, [ii])
                b = plsc.load_gather(seq_vmem, [ii + 1])
                pair_ref[pl.ds(c, _LANES)] = _mod26(a) * _VOCAB + _mod26(b)

            copies = []
            for j in range(_CHUNK // _GATHER):
                sl = pl.ds(j * _GATHER, _GATHER)
                copies.append(
                    pltpu.async_copy(
                        w2_hbm.at[pair_ref.at[sl]], out_vmem.at[sl], sem
                    )
                )
            for cp_ in copies:
                cp_.wait()

        pltpu.emit_pipeline(
            body,
            grid=(npair // _CHUNK,),
            in_specs=[pl.BlockSpec((2 * _CHUNK,), lambda i: (i,))],
            out_specs=[pl.BlockSpec((_CHUNK, 2 * _DIM), lambda i: (i, 0))],
            core_axis_name=("c", "s"),
            dimension_semantics=(pltpu.PARALLEL,),
        )(seq_hbm, out_hbm)

    return emb(seq32, w2).reshape(n, _DIM)


# speedup vs baseline: 8.4063x; 2.0299x over previous
import dataclasses
import functools

import jax
import jax.numpy as jnp
from jax.experimental import pallas as pl
from jax.experimental.pallas import tpu as pltpu
from jax.experimental.pallas import tpu_sc as plsc

_VOCAB = 26
_DIM = 64
_LANES = 16
_CHUNK = 512


def _mod26(v):
    s = (v >= 26).astype(jnp.int32)
    s += (v >= 52).astype(jnp.int32)
    s += (v >= 78).astype(jnp.int32)
    s += (v >= 104).astype(jnp.int32)
    return v - 26 * s


def kernel(seq, W):
    n = seq.shape[0]
    wt = W.T.reshape(-1)

    mesh = plsc.VectorSubcoreMesh(core_axis_name="c", subcore_axis_name="s")
    cp = pltpu.CompilerParams()
    if "needs_layout_passes" in pltpu.CompilerParams.__dataclass_fields__:
        cp = dataclasses.replace(cp, needs_layout_passes=False)

    @functools.partial(
        pl.kernel,
        out_type=jax.ShapeDtypeStruct((_DIM, n), W.dtype),
        mesh=mesh,
        scratch_types=[
            pltpu.VMEM((_VOCAB * _DIM,), jnp.float32),
            pltpu.SemaphoreType.DMA,
        ],
        compiler_params=cp,
    )
    def emb(seq_hbm, wt_hbm, out_hbm, wt_v, sem):
        pltpu.async_copy(wt_hbm, wt_v, sem).wait()

        def body(seq_vmem, out_vmem):
            @pl.loop(0, _CHUNK, step=_LANES)
            def _(c):
                sl = pl.ds(c, _LANES)
                v = _mod26(seq_vmem[sl])
                for d in range(_DIM):
                    out_vmem[d, sl] = plsc.load_gather(wt_v, [v + d * _VOCAB])

        pltpu.emit_pipeline(
            body,
            grid=(n // _CHUNK,),
            in_specs=[pl.BlockSpec((_CHUNK,), lambda i: (i,))],
            out_specs=[pl.BlockSpec((_DIM, _CHUNK), lambda i: (0, i))],
            core_axis_name=("c", "s"),
            dimension_semantics=(pltpu.PARALLEL,),
        )(seq_hbm, out_hbm)

    return emb(seq, wt).T


# speedup vs baseline: 34.1948x; 4.0677x over previous
import dataclasses
import functools

import jax
import jax.numpy as jnp
from jax.experimental import pallas as pl
from jax.experimental.pallas import tpu as pltpu
from jax.experimental.pallas import tpu_sc as plsc

_VOCAB = 26
_DIM = 64
_LANES = 16
_CHUNK = 512


def _mod26(v):
    s = (v >= 26).astype(jnp.int32)
    s += (v >= 52).astype(jnp.int32)
    s += (v >= 78).astype(jnp.int32)
    s += (v >= 104).astype(jnp.int32)
    return v - 26 * s


def kernel(seq, W):
    n = seq.shape[0]
    wt = W.T.reshape(-1)

    mesh = plsc.VectorSubcoreMesh(core_axis_name="c", subcore_axis_name="s")
    cp = pltpu.CompilerParams()
    if "needs_layout_passes" in pltpu.CompilerParams.__dataclass_fields__:
        cp = dataclasses.replace(cp, needs_layout_passes=False)

    @functools.partial(
        pl.kernel,
        out_type=jax.ShapeDtypeStruct((_DIM, n), W.dtype),
        mesh=mesh,
        scratch_types=[
            pltpu.VMEM((_VOCAB * _DIM,), jnp.float32),
            pltpu.SemaphoreType.DMA,
        ],
        compiler_params=cp,
    )
    def emb(seq_hbm, wt_hbm, out_hbm, wt_v, sem):
        pltpu.async_copy(wt_hbm, wt_v, sem).wait()

        def body(seq_vmem, out_vmem):
            @plsc.parallel_loop(0, _CHUNK, step=_LANES, unroll=2)
            def _(c):
                sl = pl.ds(c, _LANES)
                v = _mod26(seq_vmem[sl])
                for d in range(_DIM):
                    out_vmem[d, sl] = plsc.load_gather(wt_v, [v + d * _VOCAB])

        pltpu.emit_pipeline(
            body,
            grid=(n // _CHUNK,),
            in_specs=[pl.BlockSpec((_CHUNK,), lambda i: (i,))],
            out_specs=[pl.BlockSpec((_DIM, _CHUNK), lambda i: (0, i))],
            core_axis_name=("c", "s"),
            dimension_semantics=(pltpu.PARALLEL,),
        )(seq_hbm, out_hbm)

    return emb(seq, wt).T
